# Initial kernel scaffold; baseline (speedup 1.0000x reference)
import jax
import jax.numpy as jnp
from jax.experimental import pallas as pl


def kernel(x, gate_w, w1, w2):
    raise NotImplementedError("write your pallas kernel here")


# speedup vs baseline: 2.7501x; 2.7501x over previous
import jax
import jax.numpy as jnp
from jax.experimental import pallas as pl


def _router_body(x_ref, gw_ref, w_ref):
    x = x_ref[...].astype(jnp.bfloat16)
    gw = gw_ref[...].astype(jnp.bfloat16)
    logits = jax.lax.dot_general(
        x, gw, (((1,), (1,)), ((), ())),
        preferred_element_type=jnp.float32)
    E = logits.shape[1]
    eidx = jax.lax.broadcasted_iota(jnp.int32, logits.shape, 1)
    m1 = jnp.max(logits, axis=1, keepdims=True)
    i1 = jnp.min(jnp.where(logits == m1, eidx, E), axis=1, keepdims=True)
    neg = jnp.float32(jnp.finfo(jnp.float32).min)
    masked = jnp.where(eidx == i1, neg, logits)
    m2 = jnp.max(masked, axis=1, keepdims=True)
    i2 = jnp.min(jnp.where(masked == m2, eidx, E), axis=1, keepdims=True)
    wa = jax.nn.sigmoid(m1 - m2)
    wb = jax.nn.sigmoid(m2 - m1)
    w_ref[...] = jnp.where(eidx == i1, wa, 0.0) + jnp.where(eidx == i2, wb, 0.0)


def _ffn_body(x_ref, w1_ref, w2_ref, wts_ref, out_ref):
    e = pl.program_id(0)
    f = pl.program_id(1)

    @pl.when((e == 0) & (f == 0))
    def _init():
        out_ref[...] = jnp.zeros_like(out_ref)

    x = x_ref[...]
    w1 = w1_ref[0].astype(jnp.bfloat16)
    h = jax.lax.dot_general(
        x, w1, (((1,), (1,)), ((), ())),
        preferred_element_type=jnp.float32)
    h = h * jax.nn.sigmoid(h)
    hb = h.astype(jnp.bfloat16)
    w2 = w2_ref[0].astype(jnp.bfloat16)
    y = jax.lax.dot_general(
        hb, w2, (((1,), (1,)), ((), ())),
        preferred_element_type=jnp.float32)
    wts = wts_ref[...]
    eidx = jax.lax.broadcasted_iota(jnp.int32, wts.shape, 1)
    wcol = jnp.sum(jnp.where(eidx == e, wts, 0.0), axis=1, keepdims=True)
    out_ref[...] += y * wcol


def kernel(x, gate_w, w1, w2):
    B, S, H = x.shape
    E, F, _ = w1.shape
    T = B * S
    x_flat = x.reshape(T, H)

    wts = pl.pallas_call(
        _router_body,
        grid=(1,),
        in_specs=[
            pl.BlockSpec((T, H), lambda i: (0, 0)),
            pl.BlockSpec((E, H), lambda i: (0, 0)),
        ],
        out_specs=pl.BlockSpec((T, E), lambda i: (0, 0)),
        out_shape=jax.ShapeDtypeStruct((T, E), jnp.float32),
    )(x_flat, gate_w)

    xb = x_flat.astype(jnp.bfloat16)
    TF = min(512, F)
    out = pl.pallas_call(
        _ffn_body,
        grid=(E, F // TF),
        in_specs=[
            pl.BlockSpec((T, H), lambda e, f: (0, 0)),
            pl.BlockSpec((1, TF, H), lambda e, f: (e, f, 0)),
            pl.BlockSpec((1, H, TF), lambda e, f: (e, 0, f)),
            pl.BlockSpec((T, E), lambda e, f: (0, 0)),
        ],
        out_specs=pl.BlockSpec((T, H), lambda e, f: (0, 0)),
        out_shape=jax.ShapeDtypeStruct((T, H), jnp.float32),
    )(xb, w1, w2, wts)

    return out.reshape(B, S, H)
